# baseline (speedup 1.0000x reference)
import jax
import jax.numpy as jnp
from jax.experimental import pallas as pl

_BLOCK = 16384


def _rowdot_body(u_ref, v_ref, o_ref):
    o_ref[...] = jnp.sum(u_ref[...] * v_ref[...], axis=1)


def kernel(user_emb, items_emb):
    n, d = user_emb.shape
    return pl.pallas_call(
        _rowdot_body,
        grid=(pl.cdiv(n, _BLOCK),),
        in_specs=[
            pl.BlockSpec((_BLOCK, d), lambda i: (i, 0)),
            pl.BlockSpec((_BLOCK, d), lambda i: (i, 0)),
        ],
        out_specs=pl.BlockSpec((_BLOCK,), lambda i: (i,)),
        out_shape=jax.ShapeDtypeStruct((n,), jnp.float32),
    )(user_emb, items_emb)
